# baseline (speedup 1.0000x reference)
import functools

import jax
import jax.numpy as jnp
from jax import lax
from jax.experimental import pallas as pl
from jax.experimental.pallas import tpu as pltpu
from jax.experimental.pallas import tpu_sc as plsc

_TILE_V = 2048
_CHUNK = 256
_TILE_V2 = 4352
_SPLIT = 53248
_RBLK = 4096


def _pair_relayout(tableT):
    e2, vocab = tableT.shape
    nb = _SPLIT // _RBLK
    last = pl.cdiv(vocab, _RBLK) - 1

    def body(in1_ref, in2_ref, out_ref):
        out_ref[...] = jnp.concatenate(
            [in1_ref[...], in2_ref[...]], axis=0).T

    return pl.pallas_call(
        body,
        grid=(nb,),
        in_specs=[
            pl.BlockSpec((e2, _RBLK), lambda i: (0, i)),
            pl.BlockSpec((e2, _RBLK),
                         lambda i: (0, jnp.minimum(i + nb, last))),
        ],
        out_specs=pl.BlockSpec((_RBLK, 2 * e2), lambda i: (i, 0)),
        out_shape=jax.ShapeDtypeStruct((_SPLIT, 2 * e2), jnp.float32),
    )(tableT, tableT)


def _gather_rows_sc(table2, idx):
    num_rows = idx.shape[0]
    depth = table2.shape[1]
    info = plsc.get_sparse_core_info()
    num_workers = info.num_cores * info.num_subcores
    rows_per_worker = num_rows // num_workers
    mesh = plsc.VectorSubcoreMesh(core_axis_name="c", subcore_axis_name="s")

    @functools.partial(
        pl.kernel,
        out_type=jax.ShapeDtypeStruct((num_rows, depth), table2.dtype),
        mesh=mesh,
        scratch_types=[
            pltpu.VMEM((rows_per_worker,), jnp.int32),
            pltpu.VMEM((rows_per_worker, depth), table2.dtype),
            pltpu.SemaphoreType.DMA,
        ],
    )
    def gather_kernel(table_hbm, idx_hbm, out_hbm, idx_v, rows_v, sem):
        wid = lax.axis_index("s") * info.num_cores + lax.axis_index("c")
        base = wid * rows_per_worker
        pltpu.sync_copy(idx_hbm.at[pl.ds(base, rows_per_worker)], idx_v)
        pltpu.async_copy(table_hbm.at[idx_v], rows_v, sem).wait()
        pltpu.sync_copy(rows_v, out_hbm.at[pl.ds(base, rows_per_worker)])

    return gather_kernel(table2, idx)


def _bdot(a, b):
    return lax.dot_general(
        a.astype(jnp.bfloat16),
        b.astype(jnp.bfloat16),
        (((1,), (1,)), ((), ())),
        preferred_element_type=jnp.float32,
    )


def _pass1_body(vocab, batch, rows_ref, par_ref, w1d_ref, b1_ref, w2_ref,
                b2_ref, h_ref, c_ref, l_ref):
    i = pl.program_id(0)
    nv = pl.num_programs(0)
    half = rows_ref.shape[1] // 2
    pair_w = rows_ref.shape[1]

    @pl.when(i == 0)
    def _init():
        lane_hi = lax.broadcasted_iota(jnp.int32, rows_ref.shape, 1) >= half
        want_hi = par_ref[...] == 1
        sel = jnp.where(lane_hi == want_hi, rows_ref[...], 0.0)
        acc = b1_ref[...].astype(jnp.float32)
        for j in range(4):
            acc = acc + _bdot(sel[j * batch:(j + 1) * batch, :],
                              w1d_ref[pl.ds(j * pair_w, pair_w), :])
        h_ref[...] = jnp.maximum(acc, 0.0)
        l_ref[...] = jnp.zeros_like(l_ref)

    hb = h_ref[...].astype(jnp.bfloat16)
    nc = _TILE_V // _CHUNK

    def chunk_sums(masked):
        parts = []
        for k in range(nc):
            w2k = w2_ref[pl.ds(k * _CHUNK, _CHUNK), :]
            lg = (lax.dot_general(hb, w2k.astype(jnp.bfloat16),
                                  (((1,), (1,)), ((), ())),
                                  preferred_element_type=jnp.float32)
                  + b2_ref[:, k * _CHUNK:(k + 1) * _CHUNK])
            e = jnp.exp(lg)
            if masked:
                col = (i * _TILE_V + k * _CHUNK
                       + lax.broadcasted_iota(jnp.int32, (1, _CHUNK), 1))
                e = jnp.where(col < vocab, e, 0.0)
            parts.append(jnp.sum(e, axis=1, keepdims=True))
        s = parts[0]
        for p in parts[1:]:
            s = s + p
        return s

    @pl.when(i < nv - 1)
    def _accum():
        l_ref[...] += chunk_sums(False)

    @pl.when(i == nv - 1)
    def _finish():
        c_ref[...] = jnp.log(l_ref[...] + chunk_sums(True))


def _pass2_body(h_ref, w2_ref, b2_ref, c_ref, out_ref):
    b2_col = b2_ref[...].T
    logits_t = _bdot(w2_ref[...], h_ref[...]) + b2_col
    out_ref[...] = logits_t - c_ref[...]


def _mlp_log_softmax(rows, par, W1d, b1r, W2, b2r):
    nrows, pair_w = rows.shape
    batch = nrows // 4
    hidden = W1d.shape[1]
    vocab = W2.shape[0]
    nv = pl.cdiv(vocab, _TILE_V)

    h, c = pl.pallas_call(
        functools.partial(_pass1_body, vocab, batch),
        grid=(nv,),
        in_specs=[
            pl.BlockSpec((nrows, pair_w), lambda i: (0, 0)),
            pl.BlockSpec((nrows, 1), lambda i: (0, 0)),
            pl.BlockSpec((4 * pair_w, hidden), lambda i: (0, 0)),
            pl.BlockSpec((1, hidden), lambda i: (0, 0)),
            pl.BlockSpec((_TILE_V, hidden), lambda i: (i, 0)),
            pl.BlockSpec((1, _TILE_V), lambda i: (0, i)),
        ],
        out_specs=[
            pl.BlockSpec((batch, hidden), lambda i: (0, 0)),
            pl.BlockSpec((batch, 1), lambda i: (0, 0)),
        ],
        out_shape=[
            jax.ShapeDtypeStruct((batch, hidden), jnp.float32),
            jax.ShapeDtypeStruct((batch, 1), jnp.float32),
        ],
        scratch_shapes=[
            pltpu.VMEM((batch, 1), jnp.float32),
        ],
    )(rows, par, W1d, b1r, W2, b2r)

    nv2 = pl.cdiv(vocab, _TILE_V2)
    out_t = pl.pallas_call(
        _pass2_body,
        grid=(nv2,),
        in_specs=[
            pl.BlockSpec((batch, hidden), lambda i: (0, 0)),
            pl.BlockSpec((_TILE_V2, hidden), lambda i: (i, 0)),
            pl.BlockSpec((1, _TILE_V2), lambda i: (0, i)),
            pl.BlockSpec((1, batch), lambda i: (0, 0)),
        ],
        out_specs=pl.BlockSpec((_TILE_V2, batch), lambda i: (i, 0)),
        out_shape=jax.ShapeDtypeStruct((vocab, batch), jnp.float32),
        compiler_params=pltpu.CompilerParams(
            vmem_limit_bytes=50 * 1024 * 1024),
    )(h, W2, b2r, c.reshape(1, -1))
    return out_t.T


def kernel(x, table, W1, b1, W2, b2):
    batch, ctx = x.shape
    embed = table.shape[1]
    idx_t = x.T.reshape(-1).astype(jnp.int32)
    hi = (idx_t >= _SPLIT).astype(jnp.int32)
    pair_idx = idx_t - hi * _SPLIT
    parity = hi.reshape(-1, 1)
    table2 = _pair_relayout(table.T)
    rows = _gather_rows_sc(table2, pair_idx)
    w1_parts = [W1[:, j * embed:(j + 1) * embed] for j in range(ctx)]
    W1d = jnp.concatenate(
        [jnp.concatenate([p, p], axis=1) for p in w1_parts], axis=0)
    return _mlp_log_softmax(rows, parity, W1d, b1.reshape(1, -1),
                            W2, b2.reshape(1, -1))


# speedup vs baseline: 1.0276x; 1.0276x over previous
import functools

import jax
import jax.numpy as jnp
from jax import lax
from jax.experimental import pallas as pl
from jax.experimental.pallas import tpu as pltpu
from jax.experimental.pallas import tpu_sc as plsc

_TILE_V = 2048
_CHUNK = 512
_TILE_V2 = 4352
_SPLIT = 53248
_RBLK = 4096


def _pair_relayout(tableT):
    e2, vocab = tableT.shape
    nb = _SPLIT // _RBLK
    last = pl.cdiv(vocab, _RBLK) - 1

    def body(in1_ref, in2_ref, out_ref):
        out_ref[...] = jnp.concatenate(
            [in1_ref[...], in2_ref[...]], axis=0).T

    return pl.pallas_call(
        body,
        grid=(nb,),
        in_specs=[
            pl.BlockSpec((e2, _RBLK), lambda i: (0, i)),
            pl.BlockSpec((e2, _RBLK),
                         lambda i: (0, jnp.minimum(i + nb, last))),
        ],
        out_specs=pl.BlockSpec((_RBLK, 2 * e2), lambda i: (i, 0)),
        out_shape=jax.ShapeDtypeStruct((_SPLIT, 2 * e2), jnp.float32),
    )(tableT, tableT)


def _gather_rows_sc(table2, idx):
    num_rows = idx.shape[0]
    depth = table2.shape[1]
    info = plsc.get_sparse_core_info()
    num_workers = info.num_cores * info.num_subcores
    rows_per_worker = num_rows // num_workers
    mesh = plsc.VectorSubcoreMesh(core_axis_name="c", subcore_axis_name="s")

    @functools.partial(
        pl.kernel,
        out_type=jax.ShapeDtypeStruct((num_rows, depth), table2.dtype),
        mesh=mesh,
        scratch_types=[
            pltpu.VMEM((rows_per_worker,), jnp.int32),
            pltpu.VMEM((rows_per_worker, depth), table2.dtype),
            pltpu.SemaphoreType.DMA,
        ],
    )
    def gather_kernel(table_hbm, idx_hbm, out_hbm, idx_v, rows_v, sem):
        wid = lax.axis_index("s") * info.num_cores + lax.axis_index("c")
        base = wid * rows_per_worker
        pltpu.sync_copy(idx_hbm.at[pl.ds(base, rows_per_worker)], idx_v)
        pltpu.async_copy(table_hbm.at[idx_v], rows_v, sem).wait()
        pltpu.sync_copy(rows_v, out_hbm.at[pl.ds(base, rows_per_worker)])

    return gather_kernel(table2, idx)


def _bdot(a, b):
    return lax.dot_general(
        a.astype(jnp.bfloat16),
        b.astype(jnp.bfloat16),
        (((1,), (1,)), ((), ())),
        preferred_element_type=jnp.float32,
    )


def _pass1_body(vocab, batch, rows_ref, par_ref, w1d_ref, b1_ref, w2_ref,
                b2_ref, h_ref, c_ref, l_ref):
    i = pl.program_id(0)
    nv = pl.num_programs(0)
    half = rows_ref.shape[1] // 2
    pair_w = rows_ref.shape[1]

    @pl.when(i == 0)
    def _init():
        lane_hi = lax.broadcasted_iota(jnp.int32, rows_ref.shape, 1) >= half
        want_hi = par_ref[...] == 1
        sel = jnp.where(lane_hi == want_hi, rows_ref[...], 0.0)
        acc = b1_ref[...].astype(jnp.float32)
        for j in range(4):
            acc = acc + _bdot(sel[j * batch:(j + 1) * batch, :],
                              w1d_ref[pl.ds(j * pair_w, pair_w), :])
        h_ref[...] = jnp.maximum(acc, 0.0)
        l_ref[...] = jnp.zeros_like(l_ref)

    hb = h_ref[...].astype(jnp.bfloat16)
    nc = _TILE_V // _CHUNK

    def chunk_sums(masked):
        parts = []
        for k in range(nc):
            w2k = w2_ref[pl.ds(k * _CHUNK, _CHUNK), :]
            lg = (lax.dot_general(hb, w2k.astype(jnp.bfloat16),
                                  (((1,), (1,)), ((), ())),
                                  preferred_element_type=jnp.float32)
                  + b2_ref[:, k * _CHUNK:(k + 1) * _CHUNK])
            e = jnp.exp(lg)
            if masked:
                col = (i * _TILE_V + k * _CHUNK
                       + lax.broadcasted_iota(jnp.int32, (1, _CHUNK), 1))
                e = jnp.where(col < vocab, e, 0.0)
            parts.append(jnp.sum(e, axis=1, keepdims=True))
        s = parts[0]
        for p in parts[1:]:
            s = s + p
        return s

    @pl.when(i < nv - 1)
    def _accum():
        l_ref[...] += chunk_sums(False)

    @pl.when(i == nv - 1)
    def _finish():
        c_ref[...] = jnp.log(l_ref[...] + chunk_sums(True))


def _pass2_body(h_ref, w2_ref, b2_ref, c_ref, out_ref):
    b2_col = b2_ref[...].T
    logits_t = _bdot(w2_ref[...], h_ref[...]) + b2_col
    out_ref[...] = logits_t - c_ref[...]


def _mlp_log_softmax(rows, par, W1d, b1r, W2, b2r):
    nrows, pair_w = rows.shape
    batch = nrows // 4
    hidden = W1d.shape[1]
    vocab = W2.shape[0]
    nv = pl.cdiv(vocab, _TILE_V)

    h, c = pl.pallas_call(
        functools.partial(_pass1_body, vocab, batch),
        grid=(nv,),
        in_specs=[
            pl.BlockSpec((nrows, pair_w), lambda i: (0, 0)),
            pl.BlockSpec((nrows, 1), lambda i: (0, 0)),
            pl.BlockSpec((4 * pair_w, hidden), lambda i: (0, 0)),
            pl.BlockSpec((1, hidden), lambda i: (0, 0)),
            pl.BlockSpec((_TILE_V, hidden), lambda i: (i, 0)),
            pl.BlockSpec((1, _TILE_V), lambda i: (0, i)),
        ],
        out_specs=[
            pl.BlockSpec((batch, hidden), lambda i: (0, 0)),
            pl.BlockSpec((batch, 1), lambda i: (0, 0)),
        ],
        out_shape=[
            jax.ShapeDtypeStruct((batch, hidden), jnp.float32),
            jax.ShapeDtypeStruct((batch, 1), jnp.float32),
        ],
        scratch_shapes=[
            pltpu.VMEM((batch, 1), jnp.float32),
        ],
    )(rows, par, W1d, b1r, W2, b2r)

    nv2 = pl.cdiv(vocab, _TILE_V2)
    out_t = pl.pallas_call(
        _pass2_body,
        grid=(nv2,),
        in_specs=[
            pl.BlockSpec((batch, hidden), lambda i: (0, 0)),
            pl.BlockSpec((_TILE_V2, hidden), lambda i: (i, 0)),
            pl.BlockSpec((1, _TILE_V2), lambda i: (0, i)),
            pl.BlockSpec((1, batch), lambda i: (0, 0)),
        ],
        out_specs=pl.BlockSpec((_TILE_V2, batch), lambda i: (i, 0)),
        out_shape=jax.ShapeDtypeStruct((vocab, batch), jnp.float32),
        compiler_params=pltpu.CompilerParams(
            vmem_limit_bytes=50 * 1024 * 1024),
    )(h, W2, b2r, c.reshape(1, -1))
    return out_t.T


def kernel(x, table, W1, b1, W2, b2):
    batch, ctx = x.shape
    embed = table.shape[1]
    idx_t = x.T.reshape(-1).astype(jnp.int32)
    hi = (idx_t >= _SPLIT).astype(jnp.int32)
    pair_idx = idx_t - hi * _SPLIT
    parity = hi.reshape(-1, 1)
    table2 = _pair_relayout(table.T)
    rows = _gather_rows_sc(table2, pair_idx)
    w1_parts = [W1[:, j * embed:(j + 1) * embed] for j in range(ctx)]
    W1d = jnp.concatenate(
        [jnp.concatenate([p, p], axis=1) for p in w1_parts], axis=0)
    return _mlp_log_softmax(rows, parity, W1d, b1.reshape(1, -1),
                            W2, b2.reshape(1, -1))


# speedup vs baseline: 1.0610x; 1.0325x over previous
import functools

import jax
import jax.numpy as jnp
from jax import lax
from jax.experimental import pallas as pl
from jax.experimental.pallas import tpu as pltpu
from jax.experimental.pallas import tpu_sc as plsc

_TILE_V = 4096
_CHUNK = 512
_TILE_V2 = 4352
_SPLIT = 57344
_RBLK = 8192


def _pair_relayout(tableT):
    e2, vocab = tableT.shape
    nb = _SPLIT // _RBLK
    last = pl.cdiv(vocab, _RBLK) - 1

    def body(in1_ref, in2_ref, out_ref):
        out_ref[...] = jnp.concatenate(
            [in1_ref[...], in2_ref[...]], axis=0).T

    return pl.pallas_call(
        body,
        grid=(nb,),
        in_specs=[
            pl.BlockSpec((e2, _RBLK), lambda i: (0, i)),
            pl.BlockSpec((e2, _RBLK),
                         lambda i: (0, jnp.minimum(i + nb, last))),
        ],
        out_specs=pl.BlockSpec((_RBLK, 2 * e2), lambda i: (i, 0)),
        out_shape=jax.ShapeDtypeStruct((_SPLIT, 2 * e2), jnp.float32),
    )(tableT, tableT)


def _gather_rows_sc(table2, idx):
    num_rows = idx.shape[0]
    depth = table2.shape[1]
    info = plsc.get_sparse_core_info()
    num_workers = info.num_cores * info.num_subcores
    rows_per_worker = num_rows // num_workers
    mesh = plsc.VectorSubcoreMesh(core_axis_name="c", subcore_axis_name="s")

    @functools.partial(
        pl.kernel,
        out_type=jax.ShapeDtypeStruct((num_rows, depth), table2.dtype),
        mesh=mesh,
        scratch_types=[
            pltpu.VMEM((rows_per_worker,), jnp.int32),
            pltpu.VMEM((rows_per_worker, depth), table2.dtype),
            pltpu.SemaphoreType.DMA,
        ],
    )
    def gather_kernel(table_hbm, idx_hbm, out_hbm, idx_v, rows_v, sem):
        wid = lax.axis_index("s") * info.num_cores + lax.axis_index("c")
        base = wid * rows_per_worker
        pltpu.sync_copy(idx_hbm.at[pl.ds(base, rows_per_worker)], idx_v)
        pltpu.async_copy(table_hbm.at[idx_v], rows_v, sem).wait()
        pltpu.sync_copy(rows_v, out_hbm.at[pl.ds(base, rows_per_worker)])

    return gather_kernel(table2, idx)


def _bdot(a, b):
    return lax.dot_general(
        a.astype(jnp.bfloat16),
        b.astype(jnp.bfloat16),
        (((1,), (1,)), ((), ())),
        preferred_element_type=jnp.float32,
    )


def _pass1_body(vocab, batch, rows_ref, par_ref, w1d_ref, b1_ref, w2_ref,
                b2_ref, h_ref, c_ref, l_ref):
    i = pl.program_id(0)
    nv = pl.num_programs(0)
    half = rows_ref.shape[1] // 2
    pair_w = rows_ref.shape[1]

    @pl.when(i == 0)
    def _init():
        lane_hi = lax.broadcasted_iota(jnp.int32, rows_ref.shape, 1) >= half
        want_hi = par_ref[...] == 1
        sel = jnp.where(lane_hi == want_hi, rows_ref[...], 0.0)
        acc = b1_ref[...].astype(jnp.float32)
        for j in range(4):
            acc = acc + _bdot(sel[j * batch:(j + 1) * batch, :],
                              w1d_ref[pl.ds(j * pair_w, pair_w), :])
        h_ref[...] = jnp.maximum(acc, 0.0)
        l_ref[...] = jnp.zeros_like(l_ref)

    hb = h_ref[...].astype(jnp.bfloat16)
    nc = _TILE_V // _CHUNK

    def chunk_sums(masked):
        parts = []
        for k in range(nc):
            w2k = w2_ref[pl.ds(k * _CHUNK, _CHUNK), :]
            lg = (lax.dot_general(hb, w2k.astype(jnp.bfloat16),
                                  (((1,), (1,)), ((), ())),
                                  preferred_element_type=jnp.float32)
                  + b2_ref[:, k * _CHUNK:(k + 1) * _CHUNK])
            e = jnp.exp(lg)
            if masked:
                col = (i * _TILE_V + k * _CHUNK
                       + lax.broadcasted_iota(jnp.int32, (1, _CHUNK), 1))
                e = jnp.where(col < vocab, e, 0.0)
            parts.append(jnp.sum(e, axis=1, keepdims=True))
        s = parts[0]
        for p in parts[1:]:
            s = s + p
        return s

    @pl.when(i < nv - 1)
    def _accum():
        l_ref[...] += chunk_sums(False)

    @pl.when(i == nv - 1)
    def _finish():
        c_ref[...] = jnp.log(l_ref[...] + chunk_sums(True))


def _pass2_body(h_ref, w2_ref, b2_ref, c_ref, out_ref):
    b2_col = b2_ref[...].T
    logits_t = _bdot(w2_ref[...], h_ref[...]) + b2_col
    out_ref[...] = logits_t - c_ref[...]


def _mlp_log_softmax(rows, par, W1d, b1r, W2, b2r):
    nrows, pair_w = rows.shape
    batch = nrows // 4
    hidden = W1d.shape[1]
    vocab = W2.shape[0]
    nv = pl.cdiv(vocab, _TILE_V)

    h, c = pl.pallas_call(
        functools.partial(_pass1_body, vocab, batch),
        grid=(nv,),
        in_specs=[
            pl.BlockSpec((nrows, pair_w), lambda i: (0, 0)),
            pl.BlockSpec((nrows, 1), lambda i: (0, 0)),
            pl.BlockSpec((4 * pair_w, hidden), lambda i: (0, 0)),
            pl.BlockSpec((1, hidden), lambda i: (0, 0)),
            pl.BlockSpec((_TILE_V, hidden), lambda i: (i, 0)),
            pl.BlockSpec((1, _TILE_V), lambda i: (0, i)),
        ],
        out_specs=[
            pl.BlockSpec((batch, hidden), lambda i: (0, 0)),
            pl.BlockSpec((batch, 1), lambda i: (0, 0)),
        ],
        out_shape=[
            jax.ShapeDtypeStruct((batch, hidden), jnp.float32),
            jax.ShapeDtypeStruct((batch, 1), jnp.float32),
        ],
        scratch_shapes=[
            pltpu.VMEM((batch, 1), jnp.float32),
        ],
    )(rows, par, W1d, b1r, W2, b2r)

    nv2 = pl.cdiv(vocab, _TILE_V2)
    out_t = pl.pallas_call(
        _pass2_body,
        grid=(nv2,),
        in_specs=[
            pl.BlockSpec((batch, hidden), lambda i: (0, 0)),
            pl.BlockSpec((_TILE_V2, hidden), lambda i: (i, 0)),
            pl.BlockSpec((1, _TILE_V2), lambda i: (0, i)),
            pl.BlockSpec((1, batch), lambda i: (0, 0)),
        ],
        out_specs=pl.BlockSpec((_TILE_V2, batch), lambda i: (i, 0)),
        out_shape=jax.ShapeDtypeStruct((vocab, batch), jnp.float32),
        compiler_params=pltpu.CompilerParams(
            vmem_limit_bytes=50 * 1024 * 1024),
    )(h, W2, b2r, c.reshape(1, -1))
    return out_t.T


def kernel(x, table, W1, b1, W2, b2):
    batch, ctx = x.shape
    embed = table.shape[1]
    idx_t = x.T.reshape(-1).astype(jnp.int32)
    hi = (idx_t >= _SPLIT).astype(jnp.int32)
    pair_idx = idx_t - hi * _SPLIT
    parity = hi.reshape(-1, 1)
    table2 = _pair_relayout(table.T)
    rows = _gather_rows_sc(table2, pair_idx)
    w1_parts = [W1[:, j * embed:(j + 1) * embed] for j in range(ctx)]
    W1d = jnp.concatenate(
        [jnp.concatenate([p, p], axis=1) for p in w1_parts], axis=0)
    return _mlp_log_softmax(rows, parity, W1d, b1.reshape(1, -1),
                            W2, b2.reshape(1, -1))


# speedup vs baseline: 1.0643x; 1.0031x over previous
import functools

import jax
import jax.numpy as jnp
from jax import lax
from jax.experimental import pallas as pl
from jax.experimental.pallas import tpu as pltpu
from jax.experimental.pallas import tpu_sc as plsc

_TILE_V = 8192
_CHUNK = 512
_TILE_V2 = 4352
_SPLIT = 57344
_RBLK = 8192


def _pair_relayout(tableT):
    e2, vocab = tableT.shape
    nb = _SPLIT // _RBLK
    last = pl.cdiv(vocab, _RBLK) - 1

    def body(in1_ref, in2_ref, out_ref):
        out_ref[...] = jnp.concatenate(
            [in1_ref[...], in2_ref[...]], axis=0).T

    return pl.pallas_call(
        body,
        grid=(nb,),
        in_specs=[
            pl.BlockSpec((e2, _RBLK), lambda i: (0, i)),
            pl.BlockSpec((e2, _RBLK),
                         lambda i: (0, jnp.minimum(i + nb, last))),
        ],
        out_specs=pl.BlockSpec((_RBLK, 2 * e2), lambda i: (i, 0)),
        out_shape=jax.ShapeDtypeStruct((_SPLIT, 2 * e2), jnp.float32),
    )(tableT, tableT)


def _gather_rows_sc(table2, idx):
    num_rows = idx.shape[0]
    depth = table2.shape[1]
    info = plsc.get_sparse_core_info()
    num_workers = info.num_cores * info.num_subcores
    rows_per_worker = num_rows // num_workers
    mesh = plsc.VectorSubcoreMesh(core_axis_name="c", subcore_axis_name="s")

    @functools.partial(
        pl.kernel,
        out_type=jax.ShapeDtypeStruct((num_rows, depth), table2.dtype),
        mesh=mesh,
        scratch_types=[
            pltpu.VMEM((rows_per_worker,), jnp.int32),
            pltpu.VMEM((rows_per_worker, depth), table2.dtype),
            pltpu.SemaphoreType.DMA,
        ],
    )
    def gather_kernel(table_hbm, idx_hbm, out_hbm, idx_v, rows_v, sem):
        wid = lax.axis_index("s") * info.num_cores + lax.axis_index("c")
        base = wid * rows_per_worker
        pltpu.sync_copy(idx_hbm.at[pl.ds(base, rows_per_worker)], idx_v)
        pltpu.async_copy(table_hbm.at[idx_v], rows_v, sem).wait()
        pltpu.sync_copy(rows_v, out_hbm.at[pl.ds(base, rows_per_worker)])

    return gather_kernel(table2, idx)


def _bdot(a, b):
    return lax.dot_general(
        a.astype(jnp.bfloat16),
        b.astype(jnp.bfloat16),
        (((1,), (1,)), ((), ())),
        preferred_element_type=jnp.float32,
    )


def _pass1_body(vocab, batch, rows_ref, par_ref, w1d_ref, b1_ref, w2_ref,
                b2_ref, h_ref, c_ref, l_ref):
    i = pl.program_id(0)
    nv = pl.num_programs(0)
    half = rows_ref.shape[1] // 2
    pair_w = rows_ref.shape[1]

    @pl.when(i == 0)
    def _init():
        lane_hi = lax.broadcasted_iota(jnp.int32, rows_ref.shape, 1) >= half
        want_hi = par_ref[...] == 1
        sel = jnp.where(lane_hi == want_hi, rows_ref[...], 0.0)
        acc = b1_ref[...].astype(jnp.float32)
        for j in range(4):
            acc = acc + _bdot(sel[j * batch:(j + 1) * batch, :],
                              w1d_ref[pl.ds(j * pair_w, pair_w), :])
        h_ref[...] = jnp.maximum(acc, 0.0)
        l_ref[...] = jnp.zeros_like(l_ref)

    hb = h_ref[...].astype(jnp.bfloat16)
    nc = _TILE_V // _CHUNK

    def chunk_sums(masked):
        parts = []
        for k in range(nc):
            w2k = w2_ref[pl.ds(k * _CHUNK, _CHUNK), :]
            lg = (lax.dot_general(hb, w2k.astype(jnp.bfloat16),
                                  (((1,), (1,)), ((), ())),
                                  preferred_element_type=jnp.float32)
                  + b2_ref[:, k * _CHUNK:(k + 1) * _CHUNK])
            e = jnp.exp(lg)
            if masked:
                col = (i * _TILE_V + k * _CHUNK
                       + lax.broadcasted_iota(jnp.int32, (1, _CHUNK), 1))
                e = jnp.where(col < vocab, e, 0.0)
            parts.append(jnp.sum(e, axis=1, keepdims=True))
        s = parts[0]
        for p in parts[1:]:
            s = s + p
        return s

    @pl.when(i < nv - 1)
    def _accum():
        l_ref[...] += chunk_sums(False)

    @pl.when(i == nv - 1)
    def _finish():
        c_ref[...] = jnp.log(l_ref[...] + chunk_sums(True))


def _pass2_body(h_ref, w2_ref, b2_ref, c_ref, out_ref):
    b2_col = b2_ref[...].T
    logits_t = _bdot(w2_ref[...], h_ref[...]) + b2_col
    out_ref[...] = logits_t - c_ref[...]


def _mlp_log_softmax(rows, par, W1d, b1r, W2, b2r):
    nrows, pair_w = rows.shape
    batch = nrows // 4
    hidden = W1d.shape[1]
    vocab = W2.shape[0]
    nv = pl.cdiv(vocab, _TILE_V)

    h, c = pl.pallas_call(
        functools.partial(_pass1_body, vocab, batch),
        grid=(nv,),
        in_specs=[
            pl.BlockSpec((nrows, pair_w), lambda i: (0, 0)),
            pl.BlockSpec((nrows, 1), lambda i: (0, 0)),
            pl.BlockSpec((4 * pair_w, hidden), lambda i: (0, 0)),
            pl.BlockSpec((1, hidden), lambda i: (0, 0)),
            pl.BlockSpec((_TILE_V, hidden), lambda i: (i, 0)),
            pl.BlockSpec((1, _TILE_V), lambda i: (0, i)),
        ],
        out_specs=[
            pl.BlockSpec((batch, hidden), lambda i: (0, 0)),
            pl.BlockSpec((batch, 1), lambda i: (0, 0)),
        ],
        out_shape=[
            jax.ShapeDtypeStruct((batch, hidden), jnp.float32),
            jax.ShapeDtypeStruct((batch, 1), jnp.float32),
        ],
        scratch_shapes=[
            pltpu.VMEM((batch, 1), jnp.float32),
        ],
    )(rows, par, W1d, b1r, W2, b2r)

    nv2 = pl.cdiv(vocab, _TILE_V2)
    out_t = pl.pallas_call(
        _pass2_body,
        grid=(nv2,),
        in_specs=[
            pl.BlockSpec((batch, hidden), lambda i: (0, 0)),
            pl.BlockSpec((_TILE_V2, hidden), lambda i: (i, 0)),
            pl.BlockSpec((1, _TILE_V2), lambda i: (0, i)),
            pl.BlockSpec((1, batch), lambda i: (0, 0)),
        ],
        out_specs=pl.BlockSpec((_TILE_V2, batch), lambda i: (i, 0)),
        out_shape=jax.ShapeDtypeStruct((vocab, batch), jnp.float32),
        compiler_params=pltpu.CompilerParams(
            vmem_limit_bytes=50 * 1024 * 1024),
    )(h, W2, b2r, c.reshape(1, -1))
    return out_t.T


def kernel(x, table, W1, b1, W2, b2):
    batch, ctx = x.shape
    embed = table.shape[1]
    idx_t = x.T.reshape(-1).astype(jnp.int32)
    hi = (idx_t >= _SPLIT).astype(jnp.int32)
    pair_idx = idx_t - hi * _SPLIT
    parity = hi.reshape(-1, 1)
    table2 = _pair_relayout(table.T)
    rows = _gather_rows_sc(table2, pair_idx)
    w1_parts = [W1[:, j * embed:(j + 1) * embed] for j in range(ctx)]
    W1d = jnp.concatenate(
        [jnp.concatenate([p, p], axis=1) for p in w1_parts], axis=0)
    return _mlp_log_softmax(rows, parity, W1d, b1.reshape(1, -1),
                            W2, b2.reshape(1, -1))


# speedup vs baseline: 1.0932x; 1.0272x over previous
import functools

import jax
import jax.numpy as jnp
from jax import lax
from jax.experimental import pallas as pl
from jax.experimental.pallas import tpu as pltpu
from jax.experimental.pallas import tpu_sc as plsc

_TILE_V = 8192
_CHUNK = 512
_TILE_V2 = 4352
_SPLIT = 57344
_RBLK = 8192


def _pair_relayout(tableT):
    e2, vocab = tableT.shape
    nb = _SPLIT // _RBLK
    last = pl.cdiv(vocab, _RBLK) - 1

    def body(in1_ref, in2_ref, out_ref):
        out_ref[...] = jnp.concatenate(
            [in1_ref[...], in2_ref[...]], axis=0).T

    return pl.pallas_call(
        body,
        grid=(nb,),
        in_specs=[
            pl.BlockSpec((e2, _RBLK), lambda i: (0, i)),
            pl.BlockSpec((e2, _RBLK),
                         lambda i: (0, jnp.minimum(i + nb, last))),
        ],
        out_specs=pl.BlockSpec((_RBLK, 2 * e2), lambda i: (i, 0)),
        out_shape=jax.ShapeDtypeStruct((_SPLIT, 2 * e2), jnp.float32),
    )(tableT, tableT)


def _gather_rows_sc(table2, idx):
    num_rows = idx.shape[0]
    depth = table2.shape[1]
    info = plsc.get_sparse_core_info()
    num_workers = info.num_cores * info.num_subcores
    rows_per_worker = num_rows // num_workers
    mesh = plsc.VectorSubcoreMesh(core_axis_name="c", subcore_axis_name="s")

    @functools.partial(
        pl.kernel,
        out_type=jax.ShapeDtypeStruct((num_rows, depth), table2.dtype),
        mesh=mesh,
        scratch_types=[
            pltpu.VMEM((rows_per_worker,), jnp.int32),
            pltpu.VMEM((rows_per_worker, depth), table2.dtype),
            pltpu.SemaphoreType.DMA,
        ],
    )
    def gather_kernel(table_hbm, idx_hbm, out_hbm, idx_v, rows_v, sem):
        wid = lax.axis_index("s") * info.num_cores + lax.axis_index("c")
        base = wid * rows_per_worker
        pltpu.sync_copy(idx_hbm.at[pl.ds(base, rows_per_worker)], idx_v)
        pltpu.async_copy(table_hbm.at[idx_v], rows_v, sem).wait()
        pltpu.sync_copy(rows_v, out_hbm.at[pl.ds(base, rows_per_worker)])

    return gather_kernel(table2, idx)


def _bdot(a, b):
    return lax.dot_general(
        a.astype(jnp.bfloat16),
        b.astype(jnp.bfloat16),
        (((1,), (1,)), ((), ())),
        preferred_element_type=jnp.float32,
    )


def _pass1_body(vocab, batch, rows_ref, par_ref, w1d_ref, b1_ref, w2_ref,
                b2_ref, h_ref, c_ref, w2b_ref, l_ref):
    i = pl.program_id(0)
    nv = pl.num_programs(0)
    half = rows_ref.shape[1] // 2
    pair_w = rows_ref.shape[1]

    @pl.when(i == 0)
    def _init():
        lane_hi = lax.broadcasted_iota(jnp.int32, rows_ref.shape, 1) >= half
        want_hi = par_ref[...] == 1
        sel = jnp.where(lane_hi == want_hi, rows_ref[...], 0.0)
        acc = b1_ref[...].astype(jnp.float32)
        for j in range(4):
            acc = acc + _bdot(sel[j * batch:(j + 1) * batch, :],
                              w1d_ref[pl.ds(j * pair_w, pair_w), :])
        h_ref[...] = jnp.maximum(acc, 0.0)
        l_ref[...] = jnp.zeros_like(l_ref)

    hb = h_ref[...].astype(jnp.bfloat16)
    nc = _TILE_V // _CHUNK

    def chunk_sums(masked):
        parts = []
        for k in range(nc):
            w2kb = w2_ref[pl.ds(k * _CHUNK, _CHUNK), :].astype(jnp.bfloat16)
            w2b_ref[pl.ds(k * _CHUNK, _CHUNK), :] = w2kb
            lg = (lax.dot_general(hb, w2kb,
                                  (((1,), (1,)), ((), ())),
                                  preferred_element_type=jnp.float32)
                  + b2_ref[:, k * _CHUNK:(k + 1) * _CHUNK])
            e = jnp.exp(lg)
            if masked:
                col = (i * _TILE_V + k * _CHUNK
                       + lax.broadcasted_iota(jnp.int32, (1, _CHUNK), 1))
                e = jnp.where(col < vocab, e, 0.0)
            parts.append(jnp.sum(e, axis=1, keepdims=True))
        s = parts[0]
        for p in parts[1:]:
            s = s + p
        return s

    @pl.when(i < nv - 1)
    def _accum():
        l_ref[...] += chunk_sums(False)

    @pl.when(i == nv - 1)
    def _finish():
        c_ref[...] = jnp.log(l_ref[...] + chunk_sums(True))


def _pass2_body(h_ref, w2_ref, b2_ref, c_ref, out_ref):
    b2_col = b2_ref[...].T
    logits_t = _bdot(w2_ref[...], h_ref[...]) + b2_col
    out_ref[...] = logits_t - c_ref[...]


def _mlp_log_softmax(rows, par, W1d, b1r, W2, b2r):
    nrows, pair_w = rows.shape
    batch = nrows // 4
    hidden = W1d.shape[1]
    vocab = W2.shape[0]
    nv = pl.cdiv(vocab, _TILE_V)

    h, c, w2bf = pl.pallas_call(
        functools.partial(_pass1_body, vocab, batch),
        grid=(nv,),
        in_specs=[
            pl.BlockSpec((nrows, pair_w), lambda i: (0, 0)),
            pl.BlockSpec((nrows, 1), lambda i: (0, 0)),
            pl.BlockSpec((4 * pair_w, hidden), lambda i: (0, 0)),
            pl.BlockSpec((1, hidden), lambda i: (0, 0)),
            pl.BlockSpec((_TILE_V, hidden), lambda i: (i, 0)),
            pl.BlockSpec((1, _TILE_V), lambda i: (0, i)),
        ],
        out_specs=[
            pl.BlockSpec((batch, hidden), lambda i: (0, 0)),
            pl.BlockSpec((batch, 1), lambda i: (0, 0)),
            pl.BlockSpec((_TILE_V, hidden), lambda i: (i, 0)),
        ],
        out_shape=[
            jax.ShapeDtypeStruct((batch, hidden), jnp.float32),
            jax.ShapeDtypeStruct((batch, 1), jnp.float32),
            jax.ShapeDtypeStruct((vocab, hidden), jnp.bfloat16),
        ],
        scratch_shapes=[
            pltpu.VMEM((batch, 1), jnp.float32),
        ],
    )(rows, par, W1d, b1r, W2, b2r)

    nv2 = pl.cdiv(vocab, _TILE_V2)
    out_t = pl.pallas_call(
        _pass2_body,
        grid=(nv2,),
        in_specs=[
            pl.BlockSpec((batch, hidden), lambda i: (0, 0)),
            pl.BlockSpec((_TILE_V2, hidden), lambda i: (i, 0)),
            pl.BlockSpec((1, _TILE_V2), lambda i: (0, i)),
            pl.BlockSpec((1, batch), lambda i: (0, 0)),
        ],
        out_specs=pl.BlockSpec((_TILE_V2, batch), lambda i: (i, 0)),
        out_shape=jax.ShapeDtypeStruct((vocab, batch), jnp.float32),
        compiler_params=pltpu.CompilerParams(
            vmem_limit_bytes=50 * 1024 * 1024),
    )(h, w2bf, b2r, c.reshape(1, -1))
    return out_t.T


def kernel(x, table, W1, b1, W2, b2):
    batch, ctx = x.shape
    embed = table.shape[1]
    idx_t = x.T.reshape(-1).astype(jnp.int32)
    hi = (idx_t >= _SPLIT).astype(jnp.int32)
    pair_idx = idx_t - hi * _SPLIT
    parity = hi.reshape(-1, 1)
    table2 = _pair_relayout(table.T)
    rows = _gather_rows_sc(table2, pair_idx)
    w1_parts = [W1[:, j * embed:(j + 1) * embed] for j in range(ctx)]
    W1d = jnp.concatenate(
        [jnp.concatenate([p, p], axis=1) for p in w1_parts], axis=0)
    return _mlp_log_softmax(rows, parity, W1d, b1.reshape(1, -1),
                            W2, b2.reshape(1, -1))


# speedup vs baseline: 1.0976x; 1.0040x over previous
import functools

import jax
import jax.numpy as jnp
from jax import lax
from jax.experimental import pallas as pl
from jax.experimental.pallas import tpu as pltpu
from jax.experimental.pallas import tpu_sc as plsc

_TILE_V = 8192
_CHUNK = 1024
_TILE_V2 = 4352
_SPLIT = 57344
_RBLK = 8192


def _pair_relayout(tableT):
    e2, vocab = tableT.shape
    nb = _SPLIT // _RBLK
    last = pl.cdiv(vocab, _RBLK) - 1

    def body(in1_ref, in2_ref, out_ref):
        out_ref[...] = jnp.concatenate(
            [in1_ref[...], in2_ref[...]], axis=0).T

    return pl.pallas_call(
        body,
        grid=(nb,),
        in_specs=[
            pl.BlockSpec((e2, _RBLK), lambda i: (0, i)),
            pl.BlockSpec((e2, _RBLK),
                         lambda i: (0, jnp.minimum(i + nb, last))),
        ],
        out_specs=pl.BlockSpec((_RBLK, 2 * e2), lambda i: (i, 0)),
        out_shape=jax.ShapeDtypeStruct((_SPLIT, 2 * e2), jnp.float32),
    )(tableT, tableT)


def _gather_rows_sc(table2, idx):
    num_rows = idx.shape[0]
    depth = table2.shape[1]
    info = plsc.get_sparse_core_info()
    num_workers = info.num_cores * info.num_subcores
    rows_per_worker = num_rows // num_workers
    mesh = plsc.VectorSubcoreMesh(core_axis_name="c", subcore_axis_name="s")

    @functools.partial(
        pl.kernel,
        out_type=jax.ShapeDtypeStruct((num_rows, depth), table2.dtype),
        mesh=mesh,
        scratch_types=[
            pltpu.VMEM((rows_per_worker,), jnp.int32),
            pltpu.VMEM((rows_per_worker, depth), table2.dtype),
            pltpu.SemaphoreType.DMA,
        ],
    )
    def gather_kernel(table_hbm, idx_hbm, out_hbm, idx_v, rows_v, sem):
        wid = lax.axis_index("s") * info.num_cores + lax.axis_index("c")
        base = wid * rows_per_worker
        pltpu.sync_copy(idx_hbm.at[pl.ds(base, rows_per_worker)], idx_v)
        pltpu.async_copy(table_hbm.at[idx_v], rows_v, sem).wait()
        pltpu.sync_copy(rows_v, out_hbm.at[pl.ds(base, rows_per_worker)])

    return gather_kernel(table2, idx)


def _bdot(a, b):
    return lax.dot_general(
        a.astype(jnp.bfloat16),
        b.astype(jnp.bfloat16),
        (((1,), (1,)), ((), ())),
        preferred_element_type=jnp.float32,
    )


def _pass1_body(vocab, batch, rows_ref, par_ref, w1d_ref, b1_ref, w2_ref,
                b2_ref, h_ref, c_ref, w2b_ref, l_ref):
    i = pl.program_id(0)
    nv = pl.num_programs(0)
    half = rows_ref.shape[1] // 2
    pair_w = rows_ref.shape[1]

    @pl.when(i == 0)
    def _init():
        lane_hi = lax.broadcasted_iota(jnp.int32, rows_ref.shape, 1) >= half
        want_hi = par_ref[...] == 1
        sel = jnp.where(lane_hi == want_hi, rows_ref[...], 0.0)
        acc = b1_ref[...].astype(jnp.float32)
        for j in range(4):
            acc = acc + _bdot(sel[j * batch:(j + 1) * batch, :],
                              w1d_ref[pl.ds(j * pair_w, pair_w), :])
        h_ref[...] = jnp.maximum(acc, 0.0)
        l_ref[...] = jnp.zeros_like(l_ref)

    hb = h_ref[...].astype(jnp.bfloat16)
    nc = _TILE_V // _CHUNK

    def chunk_sums(masked):
        parts = []
        for k in range(nc):
            w2kb = w2_ref[pl.ds(k * _CHUNK, _CHUNK), :].astype(jnp.bfloat16)
            w2b_ref[pl.ds(k * _CHUNK, _CHUNK), :] = w2kb
            lg = (lax.dot_general(hb, w2kb,
                                  (((1,), (1,)), ((), ())),
                                  preferred_element_type=jnp.float32)
                  + b2_ref[:, k * _CHUNK:(k + 1) * _CHUNK])
            e = jnp.exp(lg)
            if masked:
                col = (i * _TILE_V + k * _CHUNK
                       + lax.broadcasted_iota(jnp.int32, (1, _CHUNK), 1))
                e = jnp.where(col < vocab, e, 0.0)
            parts.append(jnp.sum(e, axis=1, keepdims=True))
        s = parts[0]
        for p in parts[1:]:
            s = s + p
        return s

    @pl.when(i < nv - 1)
    def _accum():
        l_ref[...] += chunk_sums(False)

    @pl.when(i == nv - 1)
    def _finish():
        c_ref[...] = jnp.log(l_ref[...] + chunk_sums(True))


def _pass2_body(h_ref, w2_ref, b2_ref, c_ref, out_ref):
    b2_col = b2_ref[...].T
    logits_t = _bdot(w2_ref[...], h_ref[...]) + b2_col
    out_ref[...] = logits_t - c_ref[...]


def _mlp_log_softmax(rows, par, W1d, b1r, W2, b2r):
    nrows, pair_w = rows.shape
    batch = nrows // 4
    hidden = W1d.shape[1]
    vocab = W2.shape[0]
    nv = pl.cdiv(vocab, _TILE_V)

    h, c, w2bf = pl.pallas_call(
        functools.partial(_pass1_body, vocab, batch),
        grid=(nv,),
        in_specs=[
            pl.BlockSpec((nrows, pair_w), lambda i: (0, 0)),
            pl.BlockSpec((nrows, 1), lambda i: (0, 0)),
            pl.BlockSpec((4 * pair_w, hidden), lambda i: (0, 0)),
            pl.BlockSpec((1, hidden), lambda i: (0, 0)),
            pl.BlockSpec((_TILE_V, hidden), lambda i: (i, 0)),
            pl.BlockSpec((1, _TILE_V), lambda i: (0, i)),
        ],
        out_specs=[
            pl.BlockSpec((batch, hidden), lambda i: (0, 0)),
            pl.BlockSpec((batch, 1), lambda i: (0, 0)),
            pl.BlockSpec((_TILE_V, hidden), lambda i: (i, 0)),
        ],
        out_shape=[
            jax.ShapeDtypeStruct((batch, hidden), jnp.float32),
            jax.ShapeDtypeStruct((batch, 1), jnp.float32),
            jax.ShapeDtypeStruct((vocab, hidden), jnp.bfloat16),
        ],
        scratch_shapes=[
            pltpu.VMEM((batch, 1), jnp.float32),
        ],
    )(rows, par, W1d, b1r, W2, b2r)

    nv2 = pl.cdiv(vocab, _TILE_V2)
    out_t = pl.pallas_call(
        _pass2_body,
        grid=(nv2,),
        in_specs=[
            pl.BlockSpec((batch, hidden), lambda i: (0, 0)),
            pl.BlockSpec((_TILE_V2, hidden), lambda i: (i, 0)),
            pl.BlockSpec((1, _TILE_V2), lambda i: (0, i)),
            pl.BlockSpec((1, batch), lambda i: (0, 0)),
        ],
        out_specs=pl.BlockSpec((_TILE_V2, batch), lambda i: (i, 0)),
        out_shape=jax.ShapeDtypeStruct((vocab, batch), jnp.float32),
        compiler_params=pltpu.CompilerParams(
            vmem_limit_bytes=50 * 1024 * 1024),
    )(h, w2bf, b2r, c.reshape(1, -1))
    return out_t.T


def kernel(x, table, W1, b1, W2, b2):
    batch, ctx = x.shape
    embed = table.shape[1]
    idx_t = x.T.reshape(-1).astype(jnp.int32)
    hi = (idx_t >= _SPLIT).astype(jnp.int32)
    pair_idx = idx_t - hi * _SPLIT
    parity = hi.reshape(-1, 1)
    table2 = _pair_relayout(table.T)
    rows = _gather_rows_sc(table2, pair_idx)
    w1_parts = [W1[:, j * embed:(j + 1) * embed] for j in range(ctx)]
    W1d = jnp.concatenate(
        [jnp.concatenate([p, p], axis=1) for p in w1_parts], axis=0)
    return _mlp_log_softmax(rows, parity, W1d, b1.reshape(1, -1),
                            W2, b2.reshape(1, -1))
